# baseline (speedup 1.0000x reference)
import dataclasses
import functools

import jax
import jax.numpy as jnp
from jax import lax
from jax.experimental import pallas as pl
from jax.experimental.pallas import tpu as pltpu
from jax.experimental.pallas import tpu_sc as plsc

F = 10000
V = 10000
E = 320000
D = 128
NC = 2
NS = 16
L = 16
DH = D // NC
EPT = E // NS
ECH = 80
NCH = EPT // ECH
UNR = 16
ROWS_PT = 624
TAIL = F - NS * ROWS_PT


def _sc_edge_aggregate(a_st, b_st, senders, receivers, edge_attr, w_row):
    mesh = plsc.VectorSubcoreMesh(core_axis_name="c", subcore_axis_name="s")
    cp = pltpu.CompilerParams()
    if "needs_layout_passes" in pltpu.CompilerParams.__dataclass_fields__:
        cp = dataclasses.replace(cp, needs_layout_passes=False)
    if "use_tc_tiling_on_sc" in pltpu.CompilerParams.__dataclass_fields__:
        cp = dataclasses.replace(cp, use_tc_tiling_on_sc=False)

    @functools.partial(
        pl.kernel,
        out_type=jax.ShapeDtypeStruct((NC, F, DH), jnp.float32),
        mesh=mesh,
        compiler_params=cp,
        scratch_types=[
            pltpu.VMEM((EPT,), jnp.int32),
            pltpu.VMEM((EPT,), jnp.int32),
            pltpu.VMEM((EPT,), jnp.float32),
            pltpu.VMEM((ECH,), jnp.int32),
            pltpu.VMEM((ECH,), jnp.int32),
            pltpu.VMEM((ECH,), jnp.int32),
            pltpu.VMEM((ECH, DH), jnp.bfloat16),
            pltpu.VMEM((ECH, DH), jnp.bfloat16),
            pltpu.VMEM((ECH, DH), jnp.bfloat16),
            pltpu.VMEM((ECH, DH), jnp.bfloat16),
            pltpu.VMEM((ECH, DH), jnp.float32),
            pltpu.VMEM((ECH, DH), jnp.float32),
            pltpu.VMEM((DH,), jnp.float32),
            pltpu.VMEM_SHARED((F, DH), jnp.float32),
            pltpu.SemaphoreType.DMA,
            pltpu.SemaphoreType.DMA,
        ],
    )
    def k(a_hbm, b_hbm, s_hbm, r_hbm, e_hbm, w_hbm, out_hbm,
          sidx, ridx, attr, radj0, radj1, ridx_c,
          abuf0, bbuf0, abuf1, bbuf1, mbuf0, mbuf1, wv, acc, sem0, sem1):
        cid = lax.axis_index("c")
        sid = lax.axis_index("s")

        pltpu.sync_copy(w_hbm.at[pl.ds(cid * DH, DH)], wv)
        ebase = sid * EPT
        pltpu.sync_copy(s_hbm.at[pl.ds(ebase, EPT)], sidx)
        pltpu.sync_copy(r_hbm.at[pl.ds(ebase, EPT)], ridx)
        pltpu.sync_copy(e_hbm.at[pl.ds(ebase, EPT)], attr)

        off = jnp.full((L,), cid * F, jnp.int32)

        @pl.loop(0, EPT // L)
        def _(i):
            sl = pl.ds(i * L, L)
            sidx[sl] = sidx[sl] + off

        zeros = jnp.zeros((L,), jnp.float32)

        @pl.loop(0, ECH * DH // L)
        def _(i):
            mbuf0[i // (DH // L), pl.ds((i % (DH // L)) * L, L)] = zeros

        for j in range(ROWS_PT // ECH):
            pltpu.sync_copy(mbuf0, acc.at[pl.ds(sid * ROWS_PT + j * ECH, ECH)])
        rrem = ROWS_PT % ECH
        if rrem:
            pltpu.sync_copy(
                mbuf0.at[pl.ds(0, rrem)],
                acc.at[pl.ds(sid * ROWS_PT + ROWS_PT - rrem, rrem)])

        @pl.when(sid == 0)
        def _():
            pltpu.sync_copy(mbuf0.at[pl.ds(0, TAIL)],
                            acc.at[pl.ds(NS * ROWS_PT, TAIL)])

        plsc.subcore_barrier()

        iota2 = lax.iota(jnp.int32, L) * 2
        w_ev = [plsc.load_gather(wv, [jnp.full((L,), g * 2 * L, jnp.int32)
                                      + iota2])
                for g in range(DH // (2 * L))]
        w_od = [plsc.load_gather(wv, [jnp.full((L,), g * 2 * L + 1, jnp.int32)
                                      + iota2])
                for g in range(DH // (2 * L))]
        col_ev = [jnp.full((L,), g * 2 * L, jnp.int32) + iota2
                  for g in range(DH // (2 * L))]
        col_od = [jnp.full((L,), g * 2 * L + 1, jnp.int32) + iota2
                  for g in range(DH // (2 * L))]

        def stage_issue(c, ab, bb, radj, sem):
            co = c * ECH
            for kk in range(ECH // L):
                sl = pl.ds(kk * L, L)
                radj[sl] = ridx[pl.ds(co + kk * L, L)] + off
            pltpu.async_copy(a_hbm.at[radj], ab, sem)
            pltpu.async_copy(b_hbm.at[sidx.at[pl.ds(co, ECH)]], bb, sem)

        def wait_g(c, ab, bb, radj, sem):
            co = c * ECH
            pltpu.make_async_copy(a_hbm.at[radj], ab, sem).wait()
            pltpu.make_async_copy(
                b_hbm.at[sidx.at[pl.ds(co, ECH)]], bb, sem).wait()

        def comp_scat(c, ab, bb, mb):
            co = c * ECH

            @pl.loop(0, ECH // UNR)
            def _(u):
                eb = u * UNR
                for du in range(UNR):
                    e = eb + du
                    av = plsc.load_gather(
                        attr, [jnp.full((L,), co + e, jnp.int32)])
                    erow = jnp.full((L,), e, jnp.int32)
                    for g in range(DH // (2 * L)):
                        sl = pl.ds(g * 2 * L, 2 * L)
                        a_ev, a_od = plsc.unpack(
                            ab[e, sl], format=plsc.PackFormat.INTERLEAVED,
                            preferred_element_type=jnp.float32)
                        b_ev, b_od = plsc.unpack(
                            bb[e, sl], format=plsc.PackFormat.INTERLEAVED,
                            preferred_element_type=jnp.float32)
                        m_ev = jnp.maximum(a_ev + b_ev + av * w_ev[g], 0.0)
                        m_od = jnp.maximum(a_od + b_od + av * w_od[g], 0.0)
                        plsc.store_scatter(mb, [erow, col_ev[g]], m_ev)
                        plsc.store_scatter(mb, [erow, col_od[g]], m_od)

            for kk in range(ECH // L):
                sl = pl.ds(kk * L, L)
                ridx_c[sl] = ridx[pl.ds(co + kk * L, L)]
            pltpu.sync_copy(mb, acc.at[ridx_c], add=True)

        stage_issue(0, abuf0, bbuf0, radj0, sem0)
        stage_issue(1, abuf1, bbuf1, radj1, sem1)

        @pl.loop(0, NCH // 2 - 1)
        def _(t):
            c0 = 2 * t
            wait_g(c0, abuf0, bbuf0, radj0, sem0)
            comp_scat(c0, abuf0, bbuf0, mbuf0)
            stage_issue(c0 + 2, abuf0, bbuf0, radj0, sem0)
            wait_g(c0 + 1, abuf1, bbuf1, radj1, sem1)
            comp_scat(c0 + 1, abuf1, bbuf1, mbuf1)
            stage_issue(c0 + 3, abuf1, bbuf1, radj1, sem1)

        wait_g(NCH - 2, abuf0, bbuf0, radj0, sem0)
        comp_scat(NCH - 2, abuf0, bbuf0, mbuf0)
        wait_g(NCH - 1, abuf1, bbuf1, radj1, sem1)
        comp_scat(NCH - 1, abuf1, bbuf1, mbuf1)

        plsc.subcore_barrier()
        pltpu.sync_copy(acc.at[pl.ds(sid * ROWS_PT, ROWS_PT)],
                        out_hbm.at[cid, pl.ds(sid * ROWS_PT, ROWS_PT)])

        @pl.when(sid == 0)
        def _():
            pltpu.sync_copy(acc.at[pl.ds(NS * ROWS_PT, TAIL)],
                            out_hbm.at[cid, pl.ds(NS * ROWS_PT, TAIL)])

    return k(a_st, b_st, senders, receivers, edge_attr, w_row)


_BLK = 2000


def _tc_precompute(x_factors, x_variables, W1, W2, b_msg):
    def body(xf_ref, xv_ref, w1_ref, w2_ref, b_ref, a_ref, b_out_ref):
        a = jnp.dot(xf_ref[...], w1_ref[...],
                    preferred_element_type=jnp.float32,
                    precision=lax.Precision.HIGHEST) + b_ref[...]
        b = jnp.dot(xv_ref[...], w2_ref[...],
                    preferred_element_type=jnp.float32,
                    precision=lax.Precision.HIGHEST)
        a_ref[0] = a[:, :DH].astype(jnp.bfloat16)
        a_ref[1] = a[:, DH:].astype(jnp.bfloat16)
        b_out_ref[0] = b[:, :DH].astype(jnp.bfloat16)
        b_out_ref[1] = b[:, DH:].astype(jnp.bfloat16)

    return pl.pallas_call(
        body,
        grid=(F // _BLK,),
        in_specs=[
            pl.BlockSpec((_BLK, D), lambda i: (i, 0)),
            pl.BlockSpec((_BLK, D), lambda i: (i, 0)),
            pl.BlockSpec((D, D), lambda i: (0, 0)),
            pl.BlockSpec((D, D), lambda i: (0, 0)),
            pl.BlockSpec((1, D), lambda i: (0, 0)),
        ],
        out_specs=[
            pl.BlockSpec((NC, _BLK, DH), lambda i: (0, i, 0)),
            pl.BlockSpec((NC, _BLK, DH), lambda i: (0, i, 0)),
        ],
        out_shape=[
            jax.ShapeDtypeStruct((NC, F, DH), jnp.bfloat16),
            jax.ShapeDtypeStruct((NC, V, DH), jnp.bfloat16),
        ],
    )(x_factors, x_variables, W1, W2, b_msg.reshape(1, D))


def _tc_combine(x_factors, partials, Wc1, Wc2, b_comb):
    def body(xf_ref, p_ref, w1_ref, w2_ref, b_ref, o_ref):
        aggr = jnp.concatenate([p_ref[0], p_ref[1]], axis=-1)
        acc = jnp.dot(xf_ref[...], w1_ref[...],
                      preferred_element_type=jnp.float32,
                      precision=lax.Precision.HIGHEST)
        acc += jnp.dot(aggr, w2_ref[...],
                       preferred_element_type=jnp.float32,
                       precision=lax.Precision.HIGHEST)
        o_ref[...] = jnp.maximum(acc + b_ref[...], 0.0)

    return pl.pallas_call(
        body,
        grid=(F // _BLK,),
        in_specs=[
            pl.BlockSpec((_BLK, D), lambda i: (i, 0)),
            pl.BlockSpec((NC, _BLK, DH), lambda i: (0, i, 0)),
            pl.BlockSpec((D, D), lambda i: (0, 0)),
            pl.BlockSpec((D, D), lambda i: (0, 0)),
            pl.BlockSpec((1, D), lambda i: (0, 0)),
        ],
        out_specs=pl.BlockSpec((_BLK, D), lambda i: (i, 0)),
        out_shape=jax.ShapeDtypeStruct((F, D), jnp.float32),
    )(x_factors, partials, Wc1, Wc2, b_comb.reshape(1, D))


def kernel(x_variables, x_factors, senders, receivers, edge_attr,
           W_msg, b_msg, W_comb, b_comb):
    W1 = W_msg[:D]
    W2 = W_msg[D:2 * D]
    w_row = W_msg[2 * D]
    A3, B3 = _tc_precompute(x_factors, x_variables, W1, W2, b_msg)
    a_st = A3.reshape(NC * F, DH)
    b_st = B3.reshape(NC * V, DH)
    partials = _sc_edge_aggregate(
        a_st, b_st,
        senders.astype(jnp.int32), receivers.astype(jnp.int32),
        edge_attr, w_row)
    return _tc_combine(x_factors, partials, W_comb[:D], W_comb[D:], b_comb)


# speedup vs baseline: 1.1319x; 1.1319x over previous
import dataclasses
import functools

import jax
import jax.numpy as jnp
from jax import lax
from jax.experimental import pallas as pl
from jax.experimental.pallas import tpu as pltpu
from jax.experimental.pallas import tpu_sc as plsc

F = 10000
V = 10000
E = 320000
D = 128
NC = 2
NS = 16
L = 16
DH = D // NC
EPT = E // NS
ECH = 80
NCH = EPT // ECH
UNR = 16
ROWS_PT = 624
TAIL = F - NS * ROWS_PT


def _sc_edge_aggregate(a_st, b_st, senders, receivers, edge_attr, w_row):
    mesh = plsc.VectorSubcoreMesh(core_axis_name="c", subcore_axis_name="s")
    cp = pltpu.CompilerParams()
    if "needs_layout_passes" in pltpu.CompilerParams.__dataclass_fields__:
        cp = dataclasses.replace(cp, needs_layout_passes=False)
    if "use_tc_tiling_on_sc" in pltpu.CompilerParams.__dataclass_fields__:
        cp = dataclasses.replace(cp, use_tc_tiling_on_sc=False)

    @functools.partial(
        pl.kernel,
        out_type=jax.ShapeDtypeStruct((NC, F, DH), jnp.bfloat16),
        mesh=mesh,
        compiler_params=cp,
        scratch_types=[
            pltpu.VMEM((EPT,), jnp.int32),
            pltpu.VMEM((EPT,), jnp.int32),
            pltpu.VMEM((EPT,), jnp.float32),
            pltpu.VMEM((ECH,), jnp.int32),
            pltpu.VMEM((ECH,), jnp.int32),
            pltpu.VMEM((ECH,), jnp.int32),
            pltpu.VMEM((ECH, DH), jnp.bfloat16),
            pltpu.VMEM((ECH, DH), jnp.bfloat16),
            pltpu.VMEM((ECH, DH), jnp.bfloat16),
            pltpu.VMEM((ECH, DH), jnp.bfloat16),
            pltpu.VMEM((ECH, DH), jnp.bfloat16),
            pltpu.VMEM((ECH, DH), jnp.bfloat16),
            pltpu.VMEM((DH,), jnp.bfloat16),
            pltpu.VMEM_SHARED((F, DH), jnp.bfloat16),
            pltpu.SemaphoreType.DMA,
            pltpu.SemaphoreType.DMA,
        ],
    )
    def k(a_hbm, b_hbm, s_hbm, r_hbm, e_hbm, w_hbm, out_hbm,
          sidx, ridx, attr, radj0, radj1, ridx_c,
          abuf0, bbuf0, abuf1, bbuf1, mbuf0, mbuf1, wv, acc, sem0, sem1):
        cid = lax.axis_index("c")
        sid = lax.axis_index("s")

        pltpu.sync_copy(w_hbm.at[pl.ds(cid * DH, DH)], wv)
        ebase = sid * EPT
        pltpu.sync_copy(s_hbm.at[pl.ds(ebase, EPT)], sidx)
        pltpu.sync_copy(r_hbm.at[pl.ds(ebase, EPT)], ridx)
        pltpu.sync_copy(e_hbm.at[pl.ds(ebase, EPT)], attr)

        off = jnp.full((L,), cid * F, jnp.int32)

        @pl.loop(0, EPT // L)
        def _(i):
            sl = pl.ds(i * L, L)
            sidx[sl] = sidx[sl] + off

        zeros = jnp.zeros((2 * L,), jnp.bfloat16)

        @pl.loop(0, ECH * DH // (2 * L))
        def _(i):
            mbuf0[i // (DH // (2 * L)),
                  pl.ds((i % (DH // (2 * L))) * 2 * L, 2 * L)] = zeros

        for j in range(ROWS_PT // ECH):
            pltpu.sync_copy(mbuf0, acc.at[pl.ds(sid * ROWS_PT + j * ECH, ECH)])
        rrem = ROWS_PT % ECH
        if rrem:
            pltpu.sync_copy(
                mbuf0.at[pl.ds(0, rrem)],
                acc.at[pl.ds(sid * ROWS_PT + ROWS_PT - rrem, rrem)])

        @pl.when(sid == 0)
        def _():
            pltpu.sync_copy(mbuf0.at[pl.ds(0, TAIL)],
                            acc.at[pl.ds(NS * ROWS_PT, TAIL)])

        plsc.subcore_barrier()

        w_regs = [wv[pl.ds(g * 2 * L, 2 * L)] for g in range(DH // (2 * L))]

        def stage_issue(c, ab, bb, radj, sem):
            co = c * ECH
            for kk in range(ECH // L):
                sl = pl.ds(kk * L, L)
                radj[sl] = ridx[pl.ds(co + kk * L, L)] + off
            pltpu.async_copy(a_hbm.at[radj], ab, sem)
            pltpu.async_copy(b_hbm.at[sidx.at[pl.ds(co, ECH)]], bb, sem)

        def wait_g(c, ab, bb, radj, sem):
            co = c * ECH
            pltpu.make_async_copy(a_hbm.at[radj], ab, sem).wait()
            pltpu.make_async_copy(
                b_hbm.at[sidx.at[pl.ds(co, ECH)]], bb, sem).wait()

        def comp_scat(c, ab, bb, mb):
            co = c * ECH

            @pl.loop(0, ECH // UNR)
            def _(u):
                eb = u * UNR
                for du in range(UNR):
                    e = eb + du
                    av = plsc.load_gather(
                        attr, [jnp.full((L,), co + e, jnp.int32)])
                    av2 = plsc.pack(av, av, format=plsc.PackFormat.INTERLEAVED)
                    for g in range(DH // (2 * L)):
                        sl = pl.ds(g * 2 * L, 2 * L)
                        m = ab[e, sl] + bb[e, sl] + av2 * w_regs[g]
                        mb[e, sl] = jnp.maximum(m, jnp.bfloat16(0.0))

            for kk in range(ECH // L):
                sl = pl.ds(kk * L, L)
                ridx_c[sl] = ridx[pl.ds(co + kk * L, L)]
            pltpu.sync_copy(mb, acc.at[ridx_c], add=True)

        stage_issue(0, abuf0, bbuf0, radj0, sem0)
        stage_issue(1, abuf1, bbuf1, radj1, sem1)

        @pl.loop(0, NCH // 2 - 1)
        def _(t):
            c0 = 2 * t
            wait_g(c0, abuf0, bbuf0, radj0, sem0)
            comp_scat(c0, abuf0, bbuf0, mbuf0)
            stage_issue(c0 + 2, abuf0, bbuf0, radj0, sem0)
            wait_g(c0 + 1, abuf1, bbuf1, radj1, sem1)
            comp_scat(c0 + 1, abuf1, bbuf1, mbuf1)
            stage_issue(c0 + 3, abuf1, bbuf1, radj1, sem1)

        wait_g(NCH - 2, abuf0, bbuf0, radj0, sem0)
        comp_scat(NCH - 2, abuf0, bbuf0, mbuf0)
        wait_g(NCH - 1, abuf1, bbuf1, radj1, sem1)
        comp_scat(NCH - 1, abuf1, bbuf1, mbuf1)

        plsc.subcore_barrier()
        pltpu.sync_copy(acc.at[pl.ds(sid * ROWS_PT, ROWS_PT)],
                        out_hbm.at[cid, pl.ds(sid * ROWS_PT, ROWS_PT)])

        @pl.when(sid == 0)
        def _():
            pltpu.sync_copy(acc.at[pl.ds(NS * ROWS_PT, TAIL)],
                            out_hbm.at[cid, pl.ds(NS * ROWS_PT, TAIL)])

    return k(a_st, b_st, senders, receivers, edge_attr, w_row)


_BLK = 2000


def _tc_precompute(x_factors, x_variables, W1, W2, b_msg):
    def body(xf_ref, xv_ref, w1_ref, w2_ref, b_ref, a_ref, b_out_ref):
        a = jnp.dot(xf_ref[...], w1_ref[...],
                    preferred_element_type=jnp.float32,
                    precision=lax.Precision.HIGHEST) + b_ref[...]
        b = jnp.dot(xv_ref[...], w2_ref[...],
                    preferred_element_type=jnp.float32,
                    precision=lax.Precision.HIGHEST)
        a_ref[0] = a[:, :DH].astype(jnp.bfloat16)
        a_ref[1] = a[:, DH:].astype(jnp.bfloat16)
        b_out_ref[0] = b[:, :DH].astype(jnp.bfloat16)
        b_out_ref[1] = b[:, DH:].astype(jnp.bfloat16)

    return pl.pallas_call(
        body,
        grid=(F // _BLK,),
        in_specs=[
            pl.BlockSpec((_BLK, D), lambda i: (i, 0)),
            pl.BlockSpec((_BLK, D), lambda i: (i, 0)),
            pl.BlockSpec((D, D), lambda i: (0, 0)),
            pl.BlockSpec((D, D), lambda i: (0, 0)),
            pl.BlockSpec((1, D), lambda i: (0, 0)),
        ],
        out_specs=[
            pl.BlockSpec((NC, _BLK, DH), lambda i: (0, i, 0)),
            pl.BlockSpec((NC, _BLK, DH), lambda i: (0, i, 0)),
        ],
        out_shape=[
            jax.ShapeDtypeStruct((NC, F, DH), jnp.bfloat16),
            jax.ShapeDtypeStruct((NC, V, DH), jnp.bfloat16),
        ],
    )(x_factors, x_variables, W1, W2, b_msg.reshape(1, D))


def _tc_combine(x_factors, partials, Wc1, Wc2, b_comb):
    def body(xf_ref, p_ref, w1_ref, w2_ref, b_ref, o_ref):
        aggr = jnp.concatenate([p_ref[0], p_ref[1]], axis=-1).astype(jnp.float32)
        acc = jnp.dot(xf_ref[...], w1_ref[...],
                      preferred_element_type=jnp.float32,
                      precision=lax.Precision.HIGHEST)
        acc += jnp.dot(aggr, w2_ref[...],
                       preferred_element_type=jnp.float32,
                       precision=lax.Precision.HIGHEST)
        o_ref[...] = jnp.maximum(acc + b_ref[...], 0.0)

    return pl.pallas_call(
        body,
        grid=(F // _BLK,),
        in_specs=[
            pl.BlockSpec((_BLK, D), lambda i: (i, 0)),
            pl.BlockSpec((NC, _BLK, DH), lambda i: (0, i, 0)),
            pl.BlockSpec((D, D), lambda i: (0, 0)),
            pl.BlockSpec((D, D), lambda i: (0, 0)),
            pl.BlockSpec((1, D), lambda i: (0, 0)),
        ],
        out_specs=pl.BlockSpec((_BLK, D), lambda i: (i, 0)),
        out_shape=jax.ShapeDtypeStruct((F, D), jnp.float32),
    )(x_factors, partials, Wc1, Wc2, b_comb.reshape(1, D))


def kernel(x_variables, x_factors, senders, receivers, edge_attr,
           W_msg, b_msg, W_comb, b_comb):
    W1 = W_msg[:D]
    W2 = W_msg[D:2 * D]
    w_row = W_msg[2 * D]
    A3, B3 = _tc_precompute(x_factors, x_variables, W1, W2, b_msg)
    a_st = A3.reshape(NC * F, DH)
    b_st = B3.reshape(NC * V, DH)
    partials = _sc_edge_aggregate(
        a_st, b_st,
        senders.astype(jnp.int32), receivers.astype(jnp.int32),
        edge_attr, w_row.astype(jnp.bfloat16))
    return _tc_combine(x_factors, partials, W_comb[:D], W_comb[D:], b_comb)


# speedup vs baseline: 1.2493x; 1.1036x over previous
import dataclasses
import functools

import jax
import jax.numpy as jnp
from jax import lax
from jax.experimental import pallas as pl
from jax.experimental.pallas import tpu as pltpu
from jax.experimental.pallas import tpu_sc as plsc

F = 10000
V = 10000
E = 320000
D = 128
NC = 2
NS = 16
L = 16
DH = D // NC
EPT = E // NS
ECH = 80
NCH = EPT // ECH
UNR = 16
ROWS_PT = 624
TAIL = F - NS * ROWS_PT


def _sc_edge_aggregate(sattr_arr, receivers, a_st, b_st, w_row):
    mesh = plsc.VectorSubcoreMesh(core_axis_name="c", subcore_axis_name="s")
    cp = pltpu.CompilerParams()
    if "needs_layout_passes" in pltpu.CompilerParams.__dataclass_fields__:
        cp = dataclasses.replace(cp, needs_layout_passes=False)
    if "use_tc_tiling_on_sc" in pltpu.CompilerParams.__dataclass_fields__:
        cp = dataclasses.replace(cp, use_tc_tiling_on_sc=False)

    @functools.partial(
        pl.kernel,
        out_type=jax.ShapeDtypeStruct((NC, F, DH), jnp.float32),
        mesh=mesh,
        compiler_params=cp,
        scratch_types=[
            pltpu.VMEM((EPT,), jnp.int32),
            pltpu.VMEM((EPT,), jnp.int32),
            pltpu.VMEM((ECH,), jnp.int32),
            pltpu.VMEM((ECH,), jnp.int32),
            pltpu.VMEM((ECH,), jnp.int32),
            pltpu.VMEM((ECH,), jnp.int32),
            pltpu.VMEM((ECH, DH), jnp.float32),
            pltpu.VMEM((ECH, DH), jnp.float32),
            pltpu.VMEM((ECH, DH), jnp.float32),
            pltpu.VMEM((ECH, DH), jnp.float32),
            pltpu.VMEM((ECH, DH), jnp.float32),
            pltpu.VMEM((ECH, DH), jnp.float32),
            pltpu.VMEM((DH,), jnp.float32),
            pltpu.VMEM_SHARED((F, DH), jnp.float32),
            pltpu.SemaphoreType.DMA,
            pltpu.SemaphoreType.DMA,
            pltpu.SemaphoreType.DMA,
            pltpu.SemaphoreType.DMA,
        ],
    )
    def k(sa_hbm, r_hbm, a_hbm, b_hbm, w_hbm, out_hbm,
          sattr, ridx, sadj0, sadj1, ridx_c0, ridx_c1,
          abuf0, bbuf0, abuf1, bbuf1, mbuf0, mbuf1,
          wv, acc, sem0, sem1, sems0, sems1):
        cid = lax.axis_index("c")
        sid = lax.axis_index("s")

        pltpu.sync_copy(w_hbm.at[pl.ds(cid * DH, DH)], wv)
        ebase = sid * EPT
        pltpu.sync_copy(sa_hbm.at[pl.ds(ebase, EPT)], sattr)
        pltpu.sync_copy(r_hbm.at[pl.ds(ebase, EPT)], ridx)

        off = jnp.full((L,), cid * F, jnp.int32)
        lo16 = jnp.full((L,), 0xFFFF, jnp.int32)
        hi16 = jnp.full((L,), -65536, jnp.int32)

        @pl.loop(0, EPT // L)
        def _(i):
            sl = pl.ds(i * L, L)
            ridx[sl] = ridx[sl] + off

        zeros = jnp.zeros((L,), jnp.float32)

        @pl.loop(0, ECH * DH // L)
        def _(i):
            abuf0[i // (DH // L), pl.ds((i % (DH // L)) * L, L)] = zeros

        for j in range(ROWS_PT // ECH):
            pltpu.sync_copy(abuf0, acc.at[pl.ds(sid * ROWS_PT + j * ECH, ECH)])
        rrem = ROWS_PT % ECH
        if rrem:
            pltpu.sync_copy(
                abuf0.at[pl.ds(0, rrem)],
                acc.at[pl.ds(sid * ROWS_PT + ROWS_PT - rrem, rrem)])

        @pl.when(sid == 0)
        def _():
            pltpu.sync_copy(abuf0.at[pl.ds(0, TAIL)],
                            acc.at[pl.ds(NS * ROWS_PT, TAIL)])

        plsc.subcore_barrier()

        w_regs = [wv[pl.ds(g * L, L)] for g in range(DH // L)]

        def stage_issue(c, ab, bb, sadj, sem):
            co = c * ECH
            for kk in range(ECH // L):
                sl = pl.ds(kk * L, L)
                sadj[sl] = (sattr[pl.ds(co + kk * L, L)] & lo16) + off
            pltpu.async_copy(a_hbm.at[ridx.at[pl.ds(co, ECH)]], ab, sem)
            pltpu.async_copy(b_hbm.at[sadj], bb, sem)

        def wait_g(c, ab, bb, sadj, sem):
            co = c * ECH
            pltpu.make_async_copy(
                a_hbm.at[ridx.at[pl.ds(co, ECH)]], ab, sem).wait()
            pltpu.make_async_copy(b_hbm.at[sadj], bb, sem).wait()

        def comp_scat(c, ab, bb, mb, rc, sems):
            co = c * ECH

            @pl.loop(0, ECH // UNR)
            def _(u):
                eb = u * UNR
                for du in range(UNR):
                    e = eb + du
                    avi = plsc.load_gather(
                        sattr, [jnp.full((L,), co + e, jnp.int32)])
                    av = plsc.bitcast(avi & hi16, jnp.float32)
                    for g in range(DH // L):
                        sl = pl.ds(g * L, L)
                        m = ab[e, sl] + bb[e, sl] + av * w_regs[g]
                        mb[e, sl] = jnp.maximum(m, 0.0)

            for kk in range(ECH // L):
                sl = pl.ds(kk * L, L)
                rc[sl] = ridx[pl.ds(co + kk * L, L)] - off
            pltpu.sync_copy(mb, acc.at[rc], add=True)

        def drain_scat(mb, rc, sems):
            pass

        stage_issue(0, abuf0, bbuf0, sadj0, sem0)
        stage_issue(1, abuf1, bbuf1, sadj1, sem1)
        wait_g(0, abuf0, bbuf0, sadj0, sem0)
        comp_scat(0, abuf0, bbuf0, mbuf0, ridx_c0, sems0)
        stage_issue(2, abuf0, bbuf0, sadj0, sem0)
        wait_g(1, abuf1, bbuf1, sadj1, sem1)
        comp_scat(1, abuf1, bbuf1, mbuf1, ridx_c1, sems1)
        stage_issue(3, abuf1, bbuf1, sadj1, sem1)

        @pl.loop(0, NCH // 2 - 2)
        def _(t):
            c0 = 2 * t + 2
            wait_g(c0, abuf0, bbuf0, sadj0, sem0)
            drain_scat(mbuf0, ridx_c0, sems0)
            comp_scat(c0, abuf0, bbuf0, mbuf0, ridx_c0, sems0)
            stage_issue(c0 + 2, abuf0, bbuf0, sadj0, sem0)
            wait_g(c0 + 1, abuf1, bbuf1, sadj1, sem1)
            drain_scat(mbuf1, ridx_c1, sems1)
            comp_scat(c0 + 1, abuf1, bbuf1, mbuf1, ridx_c1, sems1)
            stage_issue(c0 + 3, abuf1, bbuf1, sadj1, sem1)

        wait_g(NCH - 2, abuf0, bbuf0, sadj0, sem0)
        drain_scat(mbuf0, ridx_c0, sems0)
        comp_scat(NCH - 2, abuf0, bbuf0, mbuf0, ridx_c0, sems0)
        wait_g(NCH - 1, abuf1, bbuf1, sadj1, sem1)
        drain_scat(mbuf1, ridx_c1, sems1)
        comp_scat(NCH - 1, abuf1, bbuf1, mbuf1, ridx_c1, sems1)
        drain_scat(mbuf0, ridx_c0, sems0)
        drain_scat(mbuf1, ridx_c1, sems1)

        plsc.subcore_barrier()
        pltpu.sync_copy(acc.at[pl.ds(sid * ROWS_PT, ROWS_PT)],
                        out_hbm.at[cid, pl.ds(sid * ROWS_PT, ROWS_PT)])

        @pl.when(sid == 0)
        def _():
            pltpu.sync_copy(acc.at[pl.ds(NS * ROWS_PT, TAIL)],
                            out_hbm.at[cid, pl.ds(NS * ROWS_PT, TAIL)])

    return k(sattr_arr, receivers, a_st, b_st, w_row)


_BLK = 2000


def _tc_precompute(x_factors, x_variables, W1, W2, b_msg):
    def body(xf_ref, xv_ref, w1_ref, w2_ref, b_ref, a_ref, b_out_ref):
        a_ref[...] = jnp.dot(xf_ref[...], w1_ref[0],
                             preferred_element_type=jnp.float32,
                             precision=lax.Precision.HIGHEST) + b_ref[0]
        b_out_ref[...] = jnp.dot(xv_ref[...], w2_ref[0],
                                 preferred_element_type=jnp.float32,
                                 precision=lax.Precision.HIGHEST)

    w1_s = W1.reshape(D, NC, DH).transpose(1, 0, 2)
    w2_s = W2.reshape(D, NC, DH).transpose(1, 0, 2)
    b_s = b_msg.reshape(NC, 1, DH)
    return pl.pallas_call(
        body,
        grid=(NC, F // _BLK),
        in_specs=[
            pl.BlockSpec((_BLK, D), lambda h, i: (i, 0)),
            pl.BlockSpec((_BLK, D), lambda h, i: (i, 0)),
            pl.BlockSpec((1, D, DH), lambda h, i: (h, 0, 0)),
            pl.BlockSpec((1, D, DH), lambda h, i: (h, 0, 0)),
            pl.BlockSpec((1, 1, DH), lambda h, i: (h, 0, 0)),
        ],
        out_specs=[
            pl.BlockSpec((_BLK, DH), lambda h, i: (h * (F // _BLK) + i, 0)),
            pl.BlockSpec((_BLK, DH), lambda h, i: (h * (V // _BLK) + i, 0)),
        ],
        out_shape=[
            jax.ShapeDtypeStruct((NC * F, DH), jnp.float32),
            jax.ShapeDtypeStruct((NC * V, DH), jnp.float32),
        ],
    )(x_factors, x_variables, w1_s, w2_s, b_s)


def _tc_combine(x_factors, partials, Wc1, Wc2, b_comb):
    def body(xf_ref, p_ref, w1_ref, w2_ref, b_ref, o_ref):
        aggr = jnp.concatenate([p_ref[0], p_ref[1]], axis=-1)
        acc = jnp.dot(xf_ref[...], w1_ref[...],
                      preferred_element_type=jnp.float32,
                      precision=lax.Precision.HIGHEST)
        acc += jnp.dot(aggr, w2_ref[...],
                       preferred_element_type=jnp.float32,
                       precision=lax.Precision.HIGHEST)
        o_ref[...] = jnp.maximum(acc + b_ref[...], 0.0)

    return pl.pallas_call(
        body,
        grid=(F // _BLK,),
        in_specs=[
            pl.BlockSpec((_BLK, D), lambda i: (i, 0)),
            pl.BlockSpec((NC, _BLK, DH), lambda i: (0, i, 0)),
            pl.BlockSpec((D, D), lambda i: (0, 0)),
            pl.BlockSpec((D, D), lambda i: (0, 0)),
            pl.BlockSpec((1, D), lambda i: (0, 0)),
        ],
        out_specs=pl.BlockSpec((_BLK, D), lambda i: (i, 0)),
        out_shape=jax.ShapeDtypeStruct((F, D), jnp.float32),
    )(x_factors, partials, Wc1, Wc2, b_comb.reshape(1, D))


def kernel(x_variables, x_factors, senders, receivers, edge_attr,
           W_msg, b_msg, W_comb, b_comb):
    W1 = W_msg[:D]
    W2 = W_msg[D:2 * D]
    w_row = W_msg[2 * D]
    a_st, b_st = _tc_precompute(x_factors, x_variables, W1, W2, b_msg)
    attr_bits = jax.lax.bitcast_convert_type(
        edge_attr.astype(jnp.bfloat16), jnp.uint16).astype(jnp.uint32) << 16
    sattr_arr = jax.lax.bitcast_convert_type(
        attr_bits | senders.astype(jnp.uint32), jnp.int32)
    partials = _sc_edge_aggregate(
        sattr_arr, receivers.astype(jnp.int32), a_st, b_st, w_row)
    return _tc_combine(x_factors, partials, W_comb[:D], W_comb[D:], b_comb)
